# Initial kernel scaffold; baseline (speedup 1.0000x reference)
import functools

import jax
import jax.numpy as jnp
from jax import lax
from jax.experimental import pallas as pl
from jax.experimental.pallas import tpu as pltpu
from jax.experimental.pallas import tpu_sc as plsc

_B = 16384
_K = 20
_D_EV = 300
_D_CLS = 64
_D_TIME = 36
_D_U = 400
_NCHUNK = _D_U // 16

_NC = 2
_NS = 16
_NW = _NC * _NS
_BW = _B // _NW
_C = 4
_STEPS = _BW // _C


def _lanes():
    return lax.iota(jnp.int32, 16)


def _load_u_regs(ev_rows, cls_rows, time_rows, r):
    li = _lanes()
    row = jnp.full((16,), r, jnp.int32)
    regs = []
    for c in range(_NCHUNK):
        d0 = c * 16
        if d0 + 16 <= _D_EV:
            regs.append(plsc.load_gather(ev_rows, [row, d0 + li]))
        elif c == 18:
            a = plsc.load_gather(ev_rows, [row, jnp.minimum(d0 + li, _D_EV - 1)])
            b = plsc.load_gather(cls_rows, [row, jnp.maximum(d0 + li - _D_EV, 0)])
            regs.append(jnp.where(li < 12, a, b))
        elif d0 + 16 <= _D_EV + _D_CLS:
            regs.append(plsc.load_gather(cls_rows, [row, d0 - _D_EV + li]))
        elif c == 22:
            a = plsc.load_gather(cls_rows, [row, jnp.minimum(d0 - _D_EV + li, _D_CLS - 1)])
            b = plsc.load_gather(time_rows, [row, jnp.maximum(d0 + li - (_D_EV + _D_CLS), 0)])
            regs.append(jnp.where(li < 12, a, b))
        else:
            regs.append(plsc.load_gather(time_rows, [row, d0 - (_D_EV + _D_CLS) + li]))
    return regs


def _dot400(u_regs, vref, vrow):
    li = _lanes()
    row = jnp.full((16,), vrow, jnp.int32)
    accs = [u_regs[c] * plsc.load_gather(vref, [row, c * 16 + li])
            for c in range(4)]
    for c in range(4, _NCHUNK):
        accs[c % 4] = accs[c % 4] + u_regs[c] * plsc.load_gather(
            vref, [row, c * 16 + li])
    return jnp.sum((accs[0] + accs[1]) + (accs[2] + accs[3]))


def _store_score(score_buf, row_g, col, s):
    li = _lanes()
    plsc.store_scatter(
        score_buf,
        [jnp.full((16,), row_g, jnp.int32), jnp.full((16,), col, jnp.int32)],
        jnp.full((16,), s, jnp.float32),
        mask=li == 0,
    )


def _sc_body(ev_i_h, cls_i_h, time_i_h, pos_i_h, neg_i_h,
             evemb, clsemb, temb, ctx, out,
             idx_ev, idx_cls, idx_time, idx_pos, idx_neg,
             ev_rows, cls_rows, time_rows, pos_rows, neg_rows,
             score_buf, sem):
    cid = lax.axis_index("c")
    sid = lax.axis_index("s")
    wid = sid * _NC + cid

    pltpu.sync_copy(ev_i_h.at[wid], idx_ev)
    pltpu.sync_copy(cls_i_h.at[wid], idx_cls)
    pltpu.sync_copy(time_i_h.at[wid], idx_time)
    pltpu.sync_copy(pos_i_h.at[wid], idx_pos)
    pltpu.sync_copy(neg_i_h.at[wid], idx_neg)

    @pl.loop(0, _STEPS)
    def _step(step):
        d1 = pltpu.async_copy(evemb.at[idx_ev.at[step]], ev_rows, sem)
        d2 = pltpu.async_copy(clsemb.at[idx_cls.at[step]], cls_rows, sem)
        d3 = pltpu.async_copy(temb.at[idx_time.at[step]], time_rows, sem)
        d4 = pltpu.async_copy(ctx.at[idx_pos.at[step]], pos_rows, sem)
        d5 = pltpu.async_copy(ctx.at[idx_neg.at[step]], neg_rows, sem)
        d1.wait()
        d2.wait()
        d3.wait()
        d4.wait()
        d5.wait()

        for r in range(_C):
            u_regs = _load_u_regs(ev_rows, cls_rows, time_rows, r)
            row_g = step * _C + r
            _store_score(score_buf, row_g, 0, _dot400(u_regs, pos_rows, r))

            @pl.loop(0, _K)
            def _neg(j):
                s = _dot400(u_regs, neg_rows, r * _K + j)
                _store_score(score_buf, row_g, 1 + j, s)

    pltpu.sync_copy(score_buf, out.at[pl.ds(wid * _BW, _BW)])


_sc_scores = pl.kernel(
    _sc_body,
    out_type=jax.ShapeDtypeStruct((_B, 1 + _K), jnp.float32),
    mesh=plsc.VectorSubcoreMesh(core_axis_name="c", subcore_axis_name="s"),
    scratch_types=[
        pltpu.VMEM((_STEPS, _C), jnp.int32),
        pltpu.VMEM((_STEPS, _C), jnp.int32),
        pltpu.VMEM((_STEPS, _C), jnp.int32),
        pltpu.VMEM((_STEPS, _C), jnp.int32),
        pltpu.VMEM((_STEPS, _C * _K), jnp.int32),
        pltpu.VMEM((_C, _D_EV), jnp.float32),
        pltpu.VMEM((_C, _D_CLS), jnp.float32),
        pltpu.VMEM((_C, _D_TIME), jnp.float32),
        pltpu.VMEM((_C, _D_U), jnp.float32),
        pltpu.VMEM((_C * _K, _D_U), jnp.float32),
        pltpu.VMEM((_BW, 1 + _K), jnp.float32),
        pltpu.SemaphoreType.DMA,
    ],
)


def _log_sigmoid(x):
    return jnp.minimum(x, 0.0) - jnp.log1p(jnp.exp(-jnp.abs(x)))


def _reduce_body(s_ref, o_ref):
    s = s_ref[...]
    pos = s[:, 0:1]
    neg = s[:, 1:]
    total = jnp.sum(_log_sigmoid(pos)) + jnp.sum(_log_sigmoid(-neg))
    o_ref[0, 0] = -total / jnp.float32(_B)


_reduce_loss = pl.pallas_call(
    _reduce_body,
    out_shape=jax.ShapeDtypeStruct((1, 1), jnp.float32),
    in_specs=[pl.BlockSpec(memory_space=pltpu.VMEM)],
    out_specs=pl.BlockSpec(memory_space=pltpu.SMEM),
)


def kernel(ev_idx, cls_idx, time_idx, pos_idx, neg_idx,
           event_emb, class_emb, time_emb, context_emb):
    ev3 = ev_idx.astype(jnp.int32).reshape(_NW, _STEPS, _C)
    cls3 = cls_idx.astype(jnp.int32).reshape(_NW, _STEPS, _C)
    time3 = time_idx.astype(jnp.int32).reshape(_NW, _STEPS, _C)
    pos3 = pos_idx.astype(jnp.int32).reshape(_NW, _STEPS, _C)
    neg3 = neg_idx.astype(jnp.int32).reshape(_NW, _STEPS, _C * _K)
    scores = _sc_scores(ev3, cls3, time3, pos3, neg3,
                        event_emb, class_emb, time_emb, context_emb)
    return _reduce_loss(scores)[0, 0]


# speedup vs baseline: 3.7496x; 3.7496x over previous
import functools

import jax
import jax.numpy as jnp
from jax import lax
from jax.experimental import pallas as pl
from jax.experimental.pallas import tpu as pltpu
from jax.experimental.pallas import tpu_sc as plsc

_B = 16384
_K = 20
_D_EV = 300
_D_CLS = 64
_D_TIME = 36
_D_U = 400
_NCHUNK = _D_U // 16

_NC = 2
_NS = 16
_NW = _NC * _NS
_BW = _B // _NW
_C = 4
_STEPS = _BW // _C


def _lanes():
    return lax.iota(jnp.int32, 16)


def _load_u_regs(ev_rows, cls_rows, time_rows, r):
    li = _lanes()
    row = jnp.full((16,), r, jnp.int32)
    regs = []
    for c in range(_NCHUNK):
        d0 = c * 16
        if d0 + 16 <= _D_EV:
            regs.append(plsc.load_gather(ev_rows, [row, d0 + li]))
        elif c == 18:
            a = plsc.load_gather(ev_rows, [row, jnp.minimum(d0 + li, _D_EV - 1)])
            b = plsc.load_gather(cls_rows, [row, jnp.maximum(d0 + li - _D_EV, 0)])
            regs.append(jnp.where(li < 12, a, b))
        elif d0 + 16 <= _D_EV + _D_CLS:
            regs.append(plsc.load_gather(cls_rows, [row, d0 - _D_EV + li]))
        elif c == 22:
            a = plsc.load_gather(cls_rows, [row, jnp.minimum(d0 - _D_EV + li, _D_CLS - 1)])
            b = plsc.load_gather(time_rows, [row, jnp.maximum(d0 + li - (_D_EV + _D_CLS), 0)])
            regs.append(jnp.where(li < 12, a, b))
        else:
            regs.append(plsc.load_gather(time_rows, [row, d0 - (_D_EV + _D_CLS) + li]))
    return regs


def _dot400(u_regs, vref, vrow):
    li = _lanes()
    row = jnp.full((16,), vrow, jnp.int32)
    accs = [u_regs[c] * plsc.load_gather(vref, [row, c * 16 + li])
            for c in range(4)]
    for c in range(4, _NCHUNK):
        accs[c % 4] = accs[c % 4] + u_regs[c] * plsc.load_gather(
            vref, [row, c * 16 + li])
    return jnp.sum((accs[0] + accs[1]) + (accs[2] + accs[3]))


def _store_score(score_buf, row_g, col, s):
    li = _lanes()
    plsc.store_scatter(
        score_buf,
        [jnp.full((16,), row_g, jnp.int32), jnp.full((16,), col, jnp.int32)],
        jnp.full((16,), s, jnp.float32),
        mask=li == 0,
    )


def _sc_body(ev_i_h, cls_i_h, time_i_h, pos_i_h, neg_i_h,
             evemb, clsemb, temb, ctx, out,
             idx_ev, idx_cls, idx_time, idx_pos, idx_neg,
             ev_rows, cls_rows, time_rows, pos_rows, neg_rows,
             score_buf, sem):
    cid = lax.axis_index("c")
    sid = lax.axis_index("s")
    wid = sid * _NC + cid

    pltpu.sync_copy(ev_i_h.at[wid], idx_ev)
    pltpu.sync_copy(cls_i_h.at[wid], idx_cls)
    pltpu.sync_copy(time_i_h.at[wid], idx_time)
    pltpu.sync_copy(pos_i_h.at[wid], idx_pos)
    pltpu.sync_copy(neg_i_h.at[wid], idx_neg)

    @pl.loop(0, _STEPS)
    def _step(step):
        d1 = pltpu.async_copy(evemb.at[idx_ev.at[step]], ev_rows, sem)
        d2 = pltpu.async_copy(clsemb.at[idx_cls.at[step]], cls_rows, sem)
        d3 = pltpu.async_copy(temb.at[idx_time.at[step]], time_rows, sem)
        d4 = pltpu.async_copy(ctx.at[idx_pos.at[step]], pos_rows, sem)
        d5 = pltpu.async_copy(ctx.at[idx_neg.at[step]], neg_rows, sem)
        d1.wait()
        d2.wait()
        d3.wait()
        d4.wait()
        d5.wait()

        for r in range(_C):
            u_regs = _load_u_regs(ev_rows, cls_rows, time_rows, r)
            row_g = step * _C + r
            _store_score(score_buf, row_g, 0, _dot400(u_regs, pos_rows, r))

            @pl.loop(0, _K)
            def _neg(j):
                s = _dot400(u_regs, neg_rows, r * _K + j)
                _store_score(score_buf, row_g, 1 + j, s)

    pltpu.sync_copy(score_buf, out.at[pl.ds(wid * _BW, _BW)])


_sc_scores = pl.kernel(
    _sc_body,
    out_type=jax.ShapeDtypeStruct((_B, 1 + _K), jnp.float32),
    mesh=plsc.VectorSubcoreMesh(core_axis_name="c", subcore_axis_name="s"),
    compiler_params=pltpu.CompilerParams(use_tc_tiling_on_sc=False,
                                         needs_layout_passes=False),
    scratch_types=[
        pltpu.VMEM((_STEPS, _C), jnp.int32),
        pltpu.VMEM((_STEPS, _C), jnp.int32),
        pltpu.VMEM((_STEPS, _C), jnp.int32),
        pltpu.VMEM((_STEPS, _C), jnp.int32),
        pltpu.VMEM((_STEPS, _C * _K), jnp.int32),
        pltpu.VMEM((_C, _D_EV), jnp.float32),
        pltpu.VMEM((_C, _D_CLS), jnp.float32),
        pltpu.VMEM((_C, _D_TIME), jnp.float32),
        pltpu.VMEM((_C, _D_U), jnp.float32),
        pltpu.VMEM((_C * _K, _D_U), jnp.float32),
        pltpu.VMEM((_BW, 1 + _K), jnp.float32),
        pltpu.SemaphoreType.DMA,
    ],
)


def _log_sigmoid(x):
    return jnp.minimum(x, 0.0) - jnp.log1p(jnp.exp(-jnp.abs(x)))


def _reduce_body(s_ref, o_ref):
    s = s_ref[...]
    pos = s[:, 0:1]
    neg = s[:, 1:]
    total = jnp.sum(_log_sigmoid(pos)) + jnp.sum(_log_sigmoid(-neg))
    o_ref[0, 0] = -total / jnp.float32(_B)


_reduce_loss = pl.pallas_call(
    _reduce_body,
    out_shape=jax.ShapeDtypeStruct((1, 1), jnp.float32),
    in_specs=[pl.BlockSpec(memory_space=pltpu.VMEM)],
    out_specs=pl.BlockSpec(memory_space=pltpu.SMEM),
)


def kernel(ev_idx, cls_idx, time_idx, pos_idx, neg_idx,
           event_emb, class_emb, time_emb, context_emb):
    ev3 = ev_idx.astype(jnp.int32).reshape(_NW, _STEPS, _C)
    cls3 = cls_idx.astype(jnp.int32).reshape(_NW, _STEPS, _C)
    time3 = time_idx.astype(jnp.int32).reshape(_NW, _STEPS, _C)
    pos3 = pos_idx.astype(jnp.int32).reshape(_NW, _STEPS, _C)
    neg3 = neg_idx.astype(jnp.int32).reshape(_NW, _STEPS, _C * _K)
    scores = _sc_scores(ev3, cls3, time3, pos3, neg3,
                        event_emb, class_emb, time_emb, context_emb)
    return _reduce_loss(scores)[0, 0]


# speedup vs baseline: 4.1137x; 1.0971x over previous
import functools

import jax
import jax.numpy as jnp
from jax import lax
from jax.experimental import pallas as pl
from jax.experimental.pallas import tpu as pltpu
from jax.experimental.pallas import tpu_sc as plsc

_B = 16384
_K = 20
_D_EV = 300
_D_CLS = 64
_D_TIME = 36
_D_U = 400
_NCHUNK = _D_U // 16

_NC = 2
_NS = 16
_NW = _NC * _NS
_BW = _B // _NW
_C = 4
_STEPS = _BW // _C


def _lanes():
    return lax.iota(jnp.int32, 16)


def _load_u_regs(ev_rows, cls_rows, time_rows, r):
    li = _lanes()
    row = jnp.full((16,), r, jnp.int32)
    regs = []
    for c in range(_NCHUNK):
        d0 = c * 16
        if d0 + 16 <= _D_EV:
            regs.append(plsc.load_gather(ev_rows, [row, d0 + li]))
        elif c == 18:
            a = plsc.load_gather(ev_rows, [row, jnp.minimum(d0 + li, _D_EV - 1)])
            b = plsc.load_gather(cls_rows, [row, jnp.maximum(d0 + li - _D_EV, 0)])
            regs.append(jnp.where(li < 12, a, b))
        elif d0 + 16 <= _D_EV + _D_CLS:
            regs.append(plsc.load_gather(cls_rows, [row, d0 - _D_EV + li]))
        elif c == 22:
            a = plsc.load_gather(cls_rows, [row, jnp.minimum(d0 - _D_EV + li, _D_CLS - 1)])
            b = plsc.load_gather(time_rows, [row, jnp.maximum(d0 + li - (_D_EV + _D_CLS), 0)])
            regs.append(jnp.where(li < 12, a, b))
        else:
            regs.append(plsc.load_gather(time_rows, [row, d0 - (_D_EV + _D_CLS) + li]))
    return regs


def _dot400(u_regs, vref, vrow):
    li = _lanes()
    row = jnp.full((16,), vrow, jnp.int32)
    accs = [u_regs[c] * plsc.load_gather(vref, [row, c * 16 + li])
            for c in range(4)]
    for c in range(4, _NCHUNK):
        accs[c % 4] = accs[c % 4] + u_regs[c] * plsc.load_gather(
            vref, [row, c * 16 + li])
    return jnp.sum((accs[0] + accs[1]) + (accs[2] + accs[3]))


def _store_score(score_buf, row_g, col, s):
    li = _lanes()
    plsc.store_scatter(
        score_buf,
        [jnp.full((16,), row_g, jnp.int32), jnp.full((16,), col, jnp.int32)],
        jnp.full((16,), s, jnp.float32),
        mask=li == 0,
    )


def _sc_body(ev_i_h, cls_i_h, time_i_h, pos_i_h, neg_i_h,
             evemb, clsemb, temb, ctx, out,
             idx_ev, idx_cls, idx_time, idx_pos, idx_neg,
             ev_rows0, cls_rows0, time_rows0, pos_rows0, neg_rows0,
             ev_rows1, cls_rows1, time_rows1, pos_rows1, neg_rows1,
             score_buf, sem0, sem1):
    cid = lax.axis_index("c")
    sid = lax.axis_index("s")
    wid = sid * _NC + cid

    pltpu.sync_copy(ev_i_h.at[wid], idx_ev)
    pltpu.sync_copy(cls_i_h.at[wid], idx_cls)
    pltpu.sync_copy(time_i_h.at[wid], idx_time)
    pltpu.sync_copy(pos_i_h.at[wid], idx_pos)
    pltpu.sync_copy(neg_i_h.at[wid], idx_neg)

    sets = (
        (ev_rows0, cls_rows0, time_rows0, pos_rows0, neg_rows0, sem0),
        (ev_rows1, cls_rows1, time_rows1, pos_rows1, neg_rows1, sem1),
    )

    def fire(step, bufs):
        ev_r, cls_r, time_r, pos_r, neg_r, sem = bufs
        pltpu.async_copy(evemb.at[idx_ev.at[step]], ev_r, sem)
        pltpu.async_copy(clsemb.at[idx_cls.at[step]], cls_r, sem)
        pltpu.async_copy(temb.at[idx_time.at[step]], time_r, sem)
        pltpu.async_copy(ctx.at[idx_pos.at[step]], pos_r, sem)
        pltpu.async_copy(ctx.at[idx_neg.at[step]], neg_r, sem)

    def drain(bufs):
        ev_r, cls_r, time_r, pos_r, neg_r, sem = bufs
        pltpu.make_async_copy(evemb.at[idx_ev.at[0]], ev_r, sem).wait()
        pltpu.make_async_copy(clsemb.at[idx_cls.at[0]], cls_r, sem).wait()
        pltpu.make_async_copy(temb.at[idx_time.at[0]], time_r, sem).wait()
        pltpu.make_async_copy(ctx.at[idx_pos.at[0]], pos_r, sem).wait()
        pltpu.make_async_copy(ctx.at[idx_neg.at[0]], neg_r, sem).wait()

    def compute(step, bufs):
        ev_r, cls_r, time_r, pos_r, neg_r, _ = bufs
        for r in range(_C):
            u_regs = _load_u_regs(ev_r, cls_r, time_r, r)
            row_g = step * _C + r
            _store_score(score_buf, row_g, 0, _dot400(u_regs, pos_r, r))

            @pl.loop(0, _K)
            def _neg(j):
                s = _dot400(u_regs, neg_r, r * _K + j)
                _store_score(score_buf, row_g, 1 + j, s)

    fire(0, sets[0])
    fire(1, sets[1])

    @pl.loop(0, _STEPS, step=2)
    def _step(i2):
        for phase in range(2):
            bufs = sets[phase]
            step = i2 + phase
            drain(bufs)
            compute(step, bufs)

            @pl.when(step + 2 < _STEPS)
            def _refire():
                fire(step + 2, bufs)

    pltpu.sync_copy(score_buf, out.at[pl.ds(wid * _BW, _BW)])


_sc_scores = pl.kernel(
    _sc_body,
    out_type=jax.ShapeDtypeStruct((_B, 1 + _K), jnp.float32),
    mesh=plsc.VectorSubcoreMesh(core_axis_name="c", subcore_axis_name="s"),
    compiler_params=pltpu.CompilerParams(use_tc_tiling_on_sc=False,
                                         needs_layout_passes=False),
    scratch_types=[
        pltpu.VMEM((_STEPS, _C), jnp.int32),
        pltpu.VMEM((_STEPS, _C), jnp.int32),
        pltpu.VMEM((_STEPS, _C), jnp.int32),
        pltpu.VMEM((_STEPS, _C), jnp.int32),
        pltpu.VMEM((_STEPS, _C * _K), jnp.int32),
        pltpu.VMEM((_C, _D_EV), jnp.float32),
        pltpu.VMEM((_C, _D_CLS), jnp.float32),
        pltpu.VMEM((_C, _D_TIME), jnp.float32),
        pltpu.VMEM((_C, _D_U), jnp.float32),
        pltpu.VMEM((_C * _K, _D_U), jnp.float32),
        pltpu.VMEM((_C, _D_EV), jnp.float32),
        pltpu.VMEM((_C, _D_CLS), jnp.float32),
        pltpu.VMEM((_C, _D_TIME), jnp.float32),
        pltpu.VMEM((_C, _D_U), jnp.float32),
        pltpu.VMEM((_C * _K, _D_U), jnp.float32),
        pltpu.VMEM((_BW, 1 + _K), jnp.float32),
        pltpu.SemaphoreType.DMA,
        pltpu.SemaphoreType.DMA,
    ],
)


def _log_sigmoid(x):
    return jnp.minimum(x, 0.0) - jnp.log1p(jnp.exp(-jnp.abs(x)))


def _reduce_body(s_ref, o_ref):
    s = s_ref[...]
    pos = s[:, 0:1]
    neg = s[:, 1:]
    total = jnp.sum(_log_sigmoid(pos)) + jnp.sum(_log_sigmoid(-neg))
    o_ref[0, 0] = -total / jnp.float32(_B)


_reduce_loss = pl.pallas_call(
    _reduce_body,
    out_shape=jax.ShapeDtypeStruct((1, 1), jnp.float32),
    in_specs=[pl.BlockSpec(memory_space=pltpu.VMEM)],
    out_specs=pl.BlockSpec(memory_space=pltpu.SMEM),
)


def kernel(ev_idx, cls_idx, time_idx, pos_idx, neg_idx,
           event_emb, class_emb, time_emb, context_emb):
    ev3 = ev_idx.astype(jnp.int32).reshape(_NW, _STEPS, _C)
    cls3 = cls_idx.astype(jnp.int32).reshape(_NW, _STEPS, _C)
    time3 = time_idx.astype(jnp.int32).reshape(_NW, _STEPS, _C)
    pos3 = pos_idx.astype(jnp.int32).reshape(_NW, _STEPS, _C)
    neg3 = neg_idx.astype(jnp.int32).reshape(_NW, _STEPS, _C * _K)
    scores = _sc_scores(ev3, cls3, time3, pos3, neg3,
                        event_emb, class_emb, time_emb, context_emb)
    return _reduce_loss(scores)[0, 0]


# speedup vs baseline: 6.5276x; 1.5868x over previous
import functools

import jax
import jax.numpy as jnp
from jax import lax
from jax.experimental import pallas as pl
from jax.experimental.pallas import tpu as pltpu
from jax.experimental.pallas import tpu_sc as plsc

_B = 16384
_K = 20
_D_EV = 300
_D_CLS = 64
_D_TIME = 36
_D_U = 400
_NCHUNK = _D_U // 16

_NC = 2
_NS = 16
_NW = _NC * _NS
_BW = _B // _NW
_C = 4
_STEPS = _BW // _C


def _lanes():
    return lax.iota(jnp.int32, 16)


def _load_u_regs(ev_rows, cls_rows, time_rows, r):
    li = _lanes()
    row = jnp.full((16,), r, jnp.int32)
    regs = []
    for c in range(_NCHUNK):
        d0 = c * 16
        if d0 + 16 <= _D_EV:
            regs.append(plsc.load_gather(ev_rows, [row, d0 + li]))
        elif c == 18:
            a = plsc.load_gather(ev_rows, [row, jnp.minimum(d0 + li, _D_EV - 1)])
            b = plsc.load_gather(cls_rows, [row, jnp.maximum(d0 + li - _D_EV, 0)])
            regs.append(jnp.where(li < 12, a, b))
        elif d0 + 16 <= _D_EV + _D_CLS:
            regs.append(plsc.load_gather(cls_rows, [row, d0 - _D_EV + li]))
        elif c == 22:
            a = plsc.load_gather(cls_rows, [row, jnp.minimum(d0 - _D_EV + li, _D_CLS - 1)])
            b = plsc.load_gather(time_rows, [row, jnp.maximum(d0 + li - (_D_EV + _D_CLS), 0)])
            regs.append(jnp.where(li < 12, a, b))
        else:
            regs.append(plsc.load_gather(time_rows, [row, d0 - (_D_EV + _D_CLS) + li]))
    return regs


def _dot400(u_regs, vref, vrow):
    li = _lanes()
    row = jnp.full((16,), vrow, jnp.int32)
    accs = [u_regs[c] * plsc.load_gather(vref, [row, c * 16 + li])
            for c in range(4)]
    for c in range(4, _NCHUNK):
        accs[c % 4] = accs[c % 4] + u_regs[c] * plsc.load_gather(
            vref, [row, c * 16 + li])
    return jnp.sum((accs[0] + accs[1]) + (accs[2] + accs[3]))


def _store_score(score_buf, row_g, col, s):
    li = _lanes()
    plsc.store_scatter(
        score_buf,
        [jnp.full((16,), row_g, jnp.int32), jnp.full((16,), col, jnp.int32)],
        jnp.full((16,), s, jnp.float32),
        mask=li == 0,
    )


def _sc_body(ev_i_h, cls_i_h, time_i_h, pos_i_h, neg_i_h,
             evemb, clsemb, temb, ctx, out,
             idx_ev, idx_cls, idx_time, idx_pos, idx_neg,
             ev_rows0, cls_rows0, time_rows0, pos_rows0, neg_rows0,
             ev_rows1, cls_rows1, time_rows1, pos_rows1, neg_rows1,
             score_buf, sem0, sem1):
    cid = lax.axis_index("c")
    sid = lax.axis_index("s")
    wid = sid * _NC + cid

    pltpu.sync_copy(ev_i_h.at[wid], idx_ev)
    pltpu.sync_copy(cls_i_h.at[wid], idx_cls)
    pltpu.sync_copy(time_i_h.at[wid], idx_time)
    pltpu.sync_copy(pos_i_h.at[wid], idx_pos)
    pltpu.sync_copy(neg_i_h.at[wid], idx_neg)

    sets = (
        (ev_rows0, cls_rows0, time_rows0, pos_rows0, neg_rows0, sem0),
        (ev_rows1, cls_rows1, time_rows1, pos_rows1, neg_rows1, sem1),
    )

    def fire(step, bufs):
        ev_r, cls_r, time_r, pos_r, neg_r, sem = bufs
        pltpu.async_copy(evemb.at[idx_ev.at[step]], ev_r, sem)
        pltpu.async_copy(clsemb.at[idx_cls.at[step]], cls_r, sem)
        pltpu.async_copy(temb.at[idx_time.at[step]], time_r, sem)
        pltpu.async_copy(ctx.at[idx_pos.at[step]], pos_r, sem)
        pltpu.async_copy(ctx.at[idx_neg.at[step]], neg_r, sem)

    def drain(bufs):
        ev_r, cls_r, time_r, pos_r, neg_r, sem = bufs
        pltpu.make_async_copy(evemb.at[idx_ev.at[0]], ev_r, sem).wait()
        pltpu.make_async_copy(clsemb.at[idx_cls.at[0]], cls_r, sem).wait()
        pltpu.make_async_copy(temb.at[idx_time.at[0]], time_r, sem).wait()
        pltpu.make_async_copy(ctx.at[idx_pos.at[0]], pos_r, sem).wait()
        pltpu.make_async_copy(ctx.at[idx_neg.at[0]], neg_r, sem).wait()

    def compute(step, bufs):
        ev_r, cls_r, time_r, pos_r, neg_r, _ = bufs
        for r in range(_C):
            u_regs = _load_u_regs(ev_r, cls_r, time_r, r)
            row_g = step * _C + r
            _store_score(score_buf, row_g, 0, _dot400(u_regs, pos_r, r))

            @pl.loop(0, _K)
            def _neg(j):
                s = _dot400(u_regs, neg_r, r * _K + j)
                _store_score(score_buf, row_g, 1 + j, s)

    fire(0, sets[0])
    fire(1, sets[1])

    @pl.loop(0, _STEPS, step=2)
    def _step(i2):
        for phase in range(2):
            bufs = sets[phase]
            step = i2 + phase
            drain(bufs)
            compute(step, bufs)

            @pl.when(step + 2 < _STEPS)
            def _refire():
                fire(step + 2, bufs)

    pltpu.sync_copy(score_buf, out.at[pl.ds(wid * _BW, _BW)])


_sc_scores = pl.kernel(
    _sc_body,
    out_type=jax.ShapeDtypeStruct((_B, 1 + _K), jnp.float32),
    mesh=plsc.VectorSubcoreMesh(core_axis_name="c", subcore_axis_name="s"),
    compiler_params=pltpu.CompilerParams(use_tc_tiling_on_sc=False,
                                         needs_layout_passes=False),
    scratch_types=[
        pltpu.VMEM((_STEPS, _C), jnp.int32),
        pltpu.VMEM((_STEPS, _C), jnp.int32),
        pltpu.VMEM((_STEPS, _C), jnp.int32),
        pltpu.VMEM((_STEPS, _C), jnp.int32),
        pltpu.VMEM((_STEPS, _C * _K), jnp.int32),
        pltpu.VMEM((_C, _D_EV), jnp.float32),
        pltpu.VMEM((_C, _D_CLS), jnp.float32),
        pltpu.VMEM((_C, _D_TIME), jnp.float32),
        pltpu.VMEM((_C, _D_U), jnp.float32),
        pltpu.VMEM((_C * _K, _D_U), jnp.float32),
        pltpu.VMEM((_C, _D_EV), jnp.float32),
        pltpu.VMEM((_C, _D_CLS), jnp.float32),
        pltpu.VMEM((_C, _D_TIME), jnp.float32),
        pltpu.VMEM((_C, _D_U), jnp.float32),
        pltpu.VMEM((_C * _K, _D_U), jnp.float32),
        pltpu.VMEM((_BW, 1 + _K), jnp.float32),
        pltpu.SemaphoreType.DMA,
        pltpu.SemaphoreType.DMA,
    ],
)


_TR_BLK = 1024


def _tr_body(x_ref, o_ref):
    o_ref[...] = x_ref[...].T


def _relayout(xt, n, d):
    grid = (n + _TR_BLK - 1) // _TR_BLK
    return pl.pallas_call(
        _tr_body,
        grid=(grid,),
        in_specs=[pl.BlockSpec((d, _TR_BLK), lambda j: (0, j))],
        out_specs=pl.BlockSpec((_TR_BLK, d), lambda j: (j, 0)),
        out_shape=jax.ShapeDtypeStruct((n, d), jnp.float32),
    )(xt)


def _log_sigmoid(x):
    return jnp.minimum(x, 0.0) - jnp.log1p(jnp.exp(-jnp.abs(x)))


def _reduce_body(s_ref, o_ref):
    s = s_ref[...]
    pos = s[:, 0:1]
    neg = s[:, 1:]
    total = jnp.sum(_log_sigmoid(pos)) + jnp.sum(_log_sigmoid(-neg))
    o_ref[0, 0] = -total / jnp.float32(_B)


_reduce_loss = pl.pallas_call(
    _reduce_body,
    out_shape=jax.ShapeDtypeStruct((1, 1), jnp.float32),
    in_specs=[pl.BlockSpec(memory_space=pltpu.VMEM)],
    out_specs=pl.BlockSpec(memory_space=pltpu.SMEM),
)


def kernel(ev_idx, cls_idx, time_idx, pos_idx, neg_idx,
           event_emb, class_emb, time_emb, context_emb):
    ev3 = ev_idx.astype(jnp.int32).reshape(_NW, _STEPS, _C)
    cls3 = cls_idx.astype(jnp.int32).reshape(_NW, _STEPS, _C)
    time3 = time_idx.astype(jnp.int32).reshape(_NW, _STEPS, _C)
    pos3 = pos_idx.astype(jnp.int32).reshape(_NW, _STEPS, _C)
    neg3 = neg_idx.astype(jnp.int32).reshape(_NW, _STEPS, _C * _K)
    event_rm = _relayout(event_emb.T, *event_emb.shape)
    ctx_rm = _relayout(context_emb.T, *context_emb.shape)
    scores = _sc_scores(ev3, cls3, time3, pos3, neg3,
                        event_rm, class_emb, time_emb, ctx_rm)
    return _reduce_loss(scores)[0, 0]


# speedup vs baseline: 11.1153x; 1.7028x over previous
import jax
import jax.numpy as jnp
from jax import lax
from jax.experimental import pallas as pl
from jax.experimental.pallas import tpu as pltpu
from jax.experimental.pallas import tpu_sc as plsc

_B = 16384
_K = 20
_D_EV = 300
_D_CLS = 64
_D_TIME = 36
_D_U = 400
_NCHUNK = _D_U // 16
_N_EV = 100000
_N_CTX = 100000

_NC = 2
_NS = 16
_NW = _NC * _NS
_BW = _B // _NW
_C = 4
_STEPS = _BW // _C

_EV_TAIL = _D_EV - 256
_CTX_TAIL = _D_U - 384


def _lanes():
    return lax.iota(jnp.int32, 16)


def _u_chunk_src(ev_g, ev_t, cls_r, time_r, c):
    d0 = c * 16
    if d0 + 16 <= 128:
        return ev_g[0], d0
    if d0 + 16 <= 256:
        return ev_g[1], d0 - 128
    if d0 + 16 <= _D_EV:
        return ev_t, d0 - 256
    if _D_EV <= d0 and d0 + 16 <= _D_EV + _D_CLS:
        return cls_r, d0 - _D_EV
    if d0 >= _D_EV + _D_CLS:
        return time_r, d0 - (_D_EV + _D_CLS)
    return None, None


def _load_u_regs(ev_g, ev_t, cls_r, time_r, r):
    li = _lanes()
    row = jnp.full((16,), r, jnp.int32)
    regs = []
    for c in range(_NCHUNK):
        d0 = c * 16
        if c == 18:
            a = plsc.load_gather(ev_t, [row, jnp.minimum(d0 - 256 + li, _EV_TAIL - 1)])
            b = plsc.load_gather(cls_r, [row, jnp.maximum(d0 + li - _D_EV, 0)])
            regs.append(jnp.where(li < 12, a, b))
        elif c == 22:
            a = plsc.load_gather(cls_r, [row, jnp.minimum(d0 - _D_EV + li, _D_CLS - 1)])
            b = plsc.load_gather(time_r, [row, jnp.maximum(d0 + li - (_D_EV + _D_CLS), 0)])
            regs.append(jnp.where(li < 12, a, b))
        else:
            ref, col0 = _u_chunk_src(ev_g, ev_t, cls_r, time_r, c)
            regs.append(plsc.load_gather(ref, [row, col0 + li]))
    return regs


def _dot400(u_regs, v_g, v_t, vrow):
    li = _lanes()
    row = jnp.full((16,), vrow, jnp.int32)

    def vchunk(c):
        if c < 24:
            return plsc.load_gather(v_g[c // 8], [row, (c % 8) * 16 + li])
        return plsc.load_gather(v_t, [row, li])

    accs = [u_regs[c] * vchunk(c) for c in range(4)]
    for c in range(4, _NCHUNK):
        accs[c % 4] = accs[c % 4] + u_regs[c] * vchunk(c)
    return jnp.sum((accs[0] + accs[1]) + (accs[2] + accs[3]))


def _store_score(score_buf, row_g, col, s):
    li = _lanes()
    plsc.store_scatter(
        score_buf,
        [jnp.full((16,), row_g, jnp.int32), jnp.full((16,), col, jnp.int32)],
        jnp.full((16,), s, jnp.float32),
        mask=li == 0,
    )


def _sc_body(ev_i_h, cls_i_h, time_i_h, pos_i_h, neg_i_h,
             ev0, ev1, evt, clsemb, temb, ctx0, ctx1, ctx2, ctxt, out,
             idx_ev, idx_cls, idx_time, idx_pos, idx_neg,
             *bufs_flat):
    score_buf = bufs_flat[-3]
    sem0 = bufs_flat[-2]
    sem1 = bufs_flat[-1]
    nper = (len(bufs_flat) - 3) // 2
    sets = (tuple(bufs_flat[:nper]) + (sem0,),
            tuple(bufs_flat[nper:2 * nper]) + (sem1,))

    cid = lax.axis_index("c")
    sid = lax.axis_index("s")
    wid = sid * _NC + cid

    pltpu.sync_copy(ev_i_h.at[wid], idx_ev)
    pltpu.sync_copy(cls_i_h.at[wid], idx_cls)
    pltpu.sync_copy(time_i_h.at[wid], idx_time)
    pltpu.sync_copy(pos_i_h.at[wid], idx_pos)
    pltpu.sync_copy(neg_i_h.at[wid], idx_neg)

    def srcs():
        return (ev0, ev1, evt, clsemb, temb, ctx0, ctx1, ctx2, ctxt,
                ctx0, ctx1, ctx2, ctxt)

    def idx_for(slot, step):
        if slot < 3:
            return idx_ev.at[step]
        if slot == 3:
            return idx_cls.at[step]
        if slot == 4:
            return idx_time.at[step]
        if slot < 9:
            return idx_pos.at[step]
        return idx_neg.at[step]

    def fire(step, bufs):
        sem = bufs[-1]
        for slot, src in enumerate(srcs()):
            pltpu.async_copy(src.at[idx_for(slot, step)], bufs[slot], sem)

    def drain(bufs):
        sem = bufs[-1]
        for slot, src in enumerate(srcs()):
            pltpu.make_async_copy(src.at[idx_for(slot, 0)], bufs[slot], sem).wait()

    def compute(step, bufs):
        (ev_g0, ev_g1, ev_t, cls_r, time_r,
         pos_g0, pos_g1, pos_g2, pos_t,
         neg_g0, neg_g1, neg_g2, neg_t, _) = bufs
        for r in range(_C):
            u_regs = _load_u_regs((ev_g0, ev_g1), ev_t, cls_r, time_r, r)
            row_g = step * _C + r
            s = _dot400(u_regs, (pos_g0, pos_g1, pos_g2), pos_t, r)
            _store_score(score_buf, row_g, 0, s)

            @pl.loop(0, _K)
            def _neg(j):
                sn = _dot400(u_regs, (neg_g0, neg_g1, neg_g2), neg_t, r * _K + j)
                _store_score(score_buf, row_g, 1 + j, sn)

    fire(0, sets[0])
    fire(1, sets[1])

    @pl.loop(0, _STEPS, step=2)
    def _step(i2):
        for phase in range(2):
            bufs = sets[phase]
            step = i2 + phase
            drain(bufs)
            compute(step, bufs)

            @pl.when(step + 2 < _STEPS)
            def _refire():
                fire(step + 2, bufs)

    pltpu.sync_copy(score_buf, out.at[pl.ds(wid * _BW, _BW)])


def _buf_set():
    return [
        pltpu.VMEM((_C, 128), jnp.float32),
        pltpu.VMEM((_C, 128), jnp.float32),
        pltpu.VMEM((_C, _EV_TAIL), jnp.float32),
        pltpu.VMEM((_C, _D_CLS), jnp.float32),
        pltpu.VMEM((_C, _D_TIME), jnp.float32),
        pltpu.VMEM((_C, 128), jnp.float32),
        pltpu.VMEM((_C, 128), jnp.float32),
        pltpu.VMEM((_C, 128), jnp.float32),
        pltpu.VMEM((_C, _CTX_TAIL), jnp.float32),
        pltpu.VMEM((_C * _K, 128), jnp.float32),
        pltpu.VMEM((_C * _K, 128), jnp.float32),
        pltpu.VMEM((_C * _K, 128), jnp.float32),
        pltpu.VMEM((_C * _K, _CTX_TAIL), jnp.float32),
    ]


_sc_scores = pl.kernel(
    _sc_body,
    out_type=jax.ShapeDtypeStruct((_B, 1 + _K), jnp.float32),
    mesh=plsc.VectorSubcoreMesh(core_axis_name="c", subcore_axis_name="s"),
    compiler_params=pltpu.CompilerParams(use_tc_tiling_on_sc=False,
                                         needs_layout_passes=False),
    scratch_types=[
        pltpu.VMEM((_STEPS, _C), jnp.int32),
        pltpu.VMEM((_STEPS, _C), jnp.int32),
        pltpu.VMEM((_STEPS, _C), jnp.int32),
        pltpu.VMEM((_STEPS, _C), jnp.int32),
        pltpu.VMEM((_STEPS, _C * _K), jnp.int32),
    ] + _buf_set() + _buf_set() + [
        pltpu.VMEM((_BW, 1 + _K), jnp.float32),
        pltpu.SemaphoreType.DMA,
        pltpu.SemaphoreType.DMA,
    ],
)


_TR_BLK = 1024


def _tr_split_body(ev_ref, ctx_ref, e0_ref, e1_ref, c0_ref, c1_ref, c2_ref):
    et = ev_ref[...].T
    e0_ref[...] = et[:, 0:128]
    e1_ref[...] = et[:, 128:256]
    ct = ctx_ref[...].T
    c0_ref[...] = ct[:, 0:128]
    c1_ref[...] = ct[:, 128:256]
    c2_ref[...] = ct[:, 256:384]


def _relayout_tables(evt_view, ctxt_view):
    grid = (_N_CTX + _TR_BLK - 1) // _TR_BLK
    return pl.pallas_call(
        _tr_split_body,
        grid=(grid,),
        in_specs=[
            pl.BlockSpec((256, _TR_BLK), lambda j: (0, j)),
            pl.BlockSpec((384, _TR_BLK), lambda j: (0, j)),
        ],
        out_specs=[pl.BlockSpec((_TR_BLK, 128), lambda j: (j, 0))] * 5,
        out_shape=[jax.ShapeDtypeStruct((_N_EV, 128), jnp.float32)] * 2
        + [jax.ShapeDtypeStruct((_N_CTX, 128), jnp.float32)] * 3,
    )(evt_view, ctxt_view)


def _log_sigmoid(x):
    return jnp.minimum(x, 0.0) - jnp.log1p(jnp.exp(-jnp.abs(x)))


def _reduce_body(s_ref, o_ref):
    s = s_ref[...]
    pos = s[:, 0:1]
    neg = s[:, 1:]
    total = jnp.sum(_log_sigmoid(pos)) + jnp.sum(_log_sigmoid(-neg))
    o_ref[0, 0] = -total / jnp.float32(_B)


_reduce_loss = pl.pallas_call(
    _reduce_body,
    out_shape=jax.ShapeDtypeStruct((1, 1), jnp.float32),
    in_specs=[pl.BlockSpec(memory_space=pltpu.VMEM)],
    out_specs=pl.BlockSpec(memory_space=pltpu.SMEM),
)


def kernel(ev_idx, cls_idx, time_idx, pos_idx, neg_idx,
           event_emb, class_emb, time_emb, context_emb):
    ev3 = ev_idx.astype(jnp.int32).reshape(_NW, _STEPS, _C)
    cls3 = cls_idx.astype(jnp.int32).reshape(_NW, _STEPS, _C)
    time3 = time_idx.astype(jnp.int32).reshape(_NW, _STEPS, _C)
    pos3 = pos_idx.astype(jnp.int32).reshape(_NW, _STEPS, _C)
    neg3 = neg_idx.astype(jnp.int32).reshape(_NW, _STEPS, _C * _K)
    ev0, ev1, c0, c1, c2 = _relayout_tables(event_emb.T, context_emb.T)
    evt = event_emb[:, 256:]
    ctxt = context_emb[:, 384:]
    scores = _sc_scores(ev3, cls3, time3, pos3, neg3,
                        ev0, ev1, evt, class_emb, time_emb,
                        c0, c1, c2, ctxt)
    return _reduce_loss(scores)[0, 0]


# speedup vs baseline: 13.9371x; 1.2539x over previous
import jax
import jax.numpy as jnp
from jax import lax
from jax.experimental import pallas as pl
from jax.experimental.pallas import tpu as pltpu
from jax.experimental.pallas import tpu_sc as plsc

_B = 16384
_K = 20
_D_EV = 300
_D_CLS = 64
_D_TIME = 36
_D_U = 400
_NCHUNK = _D_U // 16
_N_EV = 100000
_N_CTX = 100000
_NWORD_CHUNKS = 16
_NB_CHUNKS = 9

_NC = 2
_NS = 16
_NW = _NC * _NS
_BW = _B // _NW
_C = 4
_STEPS = _BW // _C


def _lanes():
    return lax.iota(jnp.int32, 16)


def _load_u_regs(ev_g, cls_r, time_r, r):
    li = _lanes()
    row = jnp.full((16,), r, jnp.int32)
    regs = []
    for c in range(_NCHUNK):
        d0 = c * 16
        if c == 18:
            a = plsc.load_gather(ev_g[2], [row, d0 - 256 + li])
            b = plsc.load_gather(cls_r, [row, jnp.maximum(d0 + li - _D_EV, 0)])
            regs.append(jnp.where(li < 12, a, b))
        elif c == 22:
            a = plsc.load_gather(cls_r, [row, jnp.minimum(d0 - _D_EV + li, _D_CLS - 1)])
            b = plsc.load_gather(time_r, [row, jnp.maximum(d0 + li - (_D_EV + _D_CLS), 0)])
            regs.append(jnp.where(li < 12, a, b))
        elif d0 + 16 <= _D_EV:
            regs.append(plsc.load_gather(ev_g[c // 8], [row, (c % 8) * 16 + li]))
        elif d0 + 16 <= _D_EV + _D_CLS:
            regs.append(plsc.load_gather(cls_r, [row, d0 - _D_EV + li]))
        else:
            regs.append(plsc.load_gather(time_r, [row, d0 - (_D_EV + _D_CLS) + li]))
    return regs


def _dot400w(u_regs, w_g, vrow):
    li = _lanes()
    row = jnp.full((16,), vrow, jnp.int32)
    accs = [jnp.zeros((16,), jnp.float32) for _ in range(4)]
    for w in range(_NWORD_CHUNKS):
        wv = plsc.load_gather(w_g[w // 8], [row, (w % 8) * 16 + li])
        wi = plsc.bitcast(wv, jnp.int32)
        a = plsc.bitcast(wi << 16, jnp.float32)
        accs[w % 4] = accs[w % 4] + u_regs[w] * a
        if w < _NB_CHUNKS:
            b = plsc.bitcast(wi & jnp.int32(-65536), jnp.float32)
            accs[(w + 2) % 4] = accs[(w + 2) % 4] + u_regs[16 + w] * b
    return jnp.sum((accs[0] + accs[1]) + (accs[2] + accs[3]))


def _store_score(score_buf, row_g, col, s):
    li = _lanes()
    plsc.store_scatter(
        score_buf,
        [jnp.full((16,), row_g, jnp.int32), jnp.full((16,), col, jnp.int32)],
        jnp.full((16,), s, jnp.float32),
        mask=li == 0,
    )


def _sc_body(ev_i_h, cls_i_h, time_i_h, pos_i_h, neg_i_h,
             ev0, ev1, ev2, clsemb, temb, ctxw0, ctxw1, out,
             idx_ev, idx_cls, idx_time, idx_pos, idx_neg,
             *bufs_flat):
    score_buf = bufs_flat[-3]
    sem0 = bufs_flat[-2]
    sem1 = bufs_flat[-1]
    nper = (len(bufs_flat) - 3) // 2
    sets = (tuple(bufs_flat[:nper]) + (sem0,),
            tuple(bufs_flat[nper:2 * nper]) + (sem1,))

    cid = lax.axis_index("c")
    sid = lax.axis_index("s")
    wid = sid * _NC + cid

    pltpu.sync_copy(ev_i_h.at[wid], idx_ev)
    pltpu.sync_copy(cls_i_h.at[wid], idx_cls)
    pltpu.sync_copy(time_i_h.at[wid], idx_time)
    pltpu.sync_copy(pos_i_h.at[wid], idx_pos)
    pltpu.sync_copy(neg_i_h.at[wid], idx_neg)

    def srcs():
        return (ev0, ev1, ev2, clsemb, temb, ctxw0, ctxw1, ctxw0, ctxw1)

    def idx_for(slot, step):
        if slot < 3:
            return idx_ev.at[step]
        if slot == 3:
            return idx_cls.at[step]
        if slot == 4:
            return idx_time.at[step]
        if slot < 7:
            return idx_pos.at[step]
        return idx_neg.at[step]

    def fire(step, bufs):
        sem = bufs[-1]
        for slot, src in enumerate(srcs()):
            pltpu.async_copy(src.at[idx_for(slot, step)], bufs[slot], sem)

    def drain(bufs):
        sem = bufs[-1]
        for slot, src in enumerate(srcs()):
            pltpu.make_async_copy(src.at[idx_for(slot, 0)], bufs[slot], sem).wait()

    def compute(step, bufs):
        (ev_g0, ev_g1, ev_g2, cls_r, time_r,
         pos_w0, pos_w1, neg_w0, neg_w1, _) = bufs
        for r in range(_C):
            u_regs = _load_u_regs((ev_g0, ev_g1, ev_g2), cls_r, time_r, r)
            row_g = step * _C + r
            s = _dot400w(u_regs, (pos_w0, pos_w1), r)
            _store_score(score_buf, row_g, 0, s)

            @pl.loop(0, _K)
            def _neg(j):
                sn = _dot400w(u_regs, (neg_w0, neg_w1), r * _K + j)
                _store_score(score_buf, row_g, 1 + j, sn)

    fire(0, sets[0])
    fire(1, sets[1])

    @pl.loop(0, _STEPS, step=2)
    def _step(i2):
        for phase in range(2):
            bufs = sets[phase]
            step = i2 + phase
            drain(bufs)
            compute(step, bufs)

            @pl.when(step + 2 < _STEPS)
            def _refire():
                fire(step + 2, bufs)

    pltpu.sync_copy(score_buf, out.at[pl.ds(wid * _BW, _BW)])


def _buf_set():
    return [
        pltpu.VMEM((_C, 128), jnp.float32),
        pltpu.VMEM((_C, 128), jnp.float32),
        pltpu.VMEM((_C, 128), jnp.float32),
        pltpu.VMEM((_C, _D_CLS), jnp.float32),
        pltpu.VMEM((_C, _D_TIME), jnp.float32),
        pltpu.VMEM((_C, 128), jnp.float32),
        pltpu.VMEM((_C, 128), jnp.float32),
        pltpu.VMEM((_C * _K, 128), jnp.float32),
        pltpu.VMEM((_C * _K, 128), jnp.float32),
    ]


_sc_scores = pl.kernel(
    _sc_body,
    out_type=jax.ShapeDtypeStruct((_B, 1 + _K), jnp.float32),
    mesh=plsc.VectorSubcoreMesh(core_axis_name="c", subcore_axis_name="s"),
    compiler_params=pltpu.CompilerParams(use_tc_tiling_on_sc=False,
                                         needs_layout_passes=False),
    scratch_types=[
        pltpu.VMEM((_STEPS, _C), jnp.int32),
        pltpu.VMEM((_STEPS, _C), jnp.int32),
        pltpu.VMEM((_STEPS, _C), jnp.int32),
        pltpu.VMEM((_STEPS, _C), jnp.int32),
        pltpu.VMEM((_STEPS, _C * _K), jnp.int32),
    ] + _buf_set() + _buf_set() + [
        pltpu.VMEM((_BW, 1 + _K), jnp.float32),
        pltpu.SemaphoreType.DMA,
        pltpu.SemaphoreType.DMA,
    ],
)


_TR_BLK = 1024


def _pack_words(xt, ncols):
    xb = xt.astype(jnp.bfloat16)
    xb = jnp.concatenate(
        [xb, jnp.zeros((_TR_BLK, 512 - ncols), jnp.bfloat16)], axis=1)
    lo = lax.convert_element_type(
        lax.bitcast_convert_type(xb[:, :256], jnp.uint16), jnp.uint32)
    hi = lax.convert_element_type(
        lax.bitcast_convert_type(xb[:, 256:], jnp.uint16), jnp.uint32)
    return lax.bitcast_convert_type(lo | (hi << 16), jnp.float32)


def _tr_split_body(ev_ref, ctx_ref, e0_ref, e1_ref, e2_ref, c0_ref, c1_ref):
    et = ev_ref[...].T
    e0_ref[...] = et[:, 0:128]
    e1_ref[...] = et[:, 128:256]
    e2_ref[...] = jnp.concatenate(
        [et[:, 256:_D_EV], jnp.zeros((_TR_BLK, 128 - (_D_EV - 256)), jnp.float32)],
        axis=1)
    cw = _pack_words(ctx_ref[...].T, _D_U)
    c0_ref[...] = cw[:, :128]
    c1_ref[...] = cw[:, 128:]


def _relayout_tables(evt_view, ctxt_view):
    grid = (_N_CTX + _TR_BLK - 1) // _TR_BLK
    return pl.pallas_call(
        _tr_split_body,
        grid=(grid,),
        in_specs=[
            pl.BlockSpec((_D_EV, _TR_BLK), lambda j: (0, j)),
            pl.BlockSpec((_D_U, _TR_BLK), lambda j: (0, j)),
        ],
        out_specs=[pl.BlockSpec((_TR_BLK, 128), lambda j: (j, 0))] * 5,
        out_shape=[jax.ShapeDtypeStruct((_N_EV, 128), jnp.float32)] * 3
        + [jax.ShapeDtypeStruct((_N_CTX, 128), jnp.float32)] * 2,
    )(evt_view, ctxt_view)


def _log_sigmoid(x):
    return jnp.minimum(x, 0.0) - jnp.log1p(jnp.exp(-jnp.abs(x)))


def _reduce_body(s_ref, o_ref):
    s = s_ref[...]
    pos = s[:, 0:1]
    neg = s[:, 1:]
    total = jnp.sum(_log_sigmoid(pos)) + jnp.sum(_log_sigmoid(-neg))
    o_ref[0, 0] = -total / jnp.float32(_B)


_reduce_loss = pl.pallas_call(
    _reduce_body,
    out_shape=jax.ShapeDtypeStruct((1, 1), jnp.float32),
    in_specs=[pl.BlockSpec(memory_space=pltpu.VMEM)],
    out_specs=pl.BlockSpec(memory_space=pltpu.SMEM),
)


def kernel(ev_idx, cls_idx, time_idx, pos_idx, neg_idx,
           event_emb, class_emb, time_emb, context_emb):
    ev3 = ev_idx.astype(jnp.int32).reshape(_NW, _STEPS, _C)
    cls3 = cls_idx.astype(jnp.int32).reshape(_NW, _STEPS, _C)
    time3 = time_idx.astype(jnp.int32).reshape(_NW, _STEPS, _C)
    pos3 = pos_idx.astype(jnp.int32).reshape(_NW, _STEPS, _C)
    neg3 = neg_idx.astype(jnp.int32).reshape(_NW, _STEPS, _C * _K)
    ev0, ev1, ev2, cw0, cw1 = _relayout_tables(event_emb.T, context_emb.T)
    scores = _sc_scores(ev3, cls3, time3, pos3, neg3,
                        ev0, ev1, ev2, class_emb, time_emb, cw0, cw1)
    return _reduce_loss(scores)[0, 0]
